# Initial kernel scaffold; baseline (speedup 1.0000x reference)
import jax
import jax.numpy as jnp
from jax.experimental import pallas as pl


def kernel(x):
    raise NotImplementedError("write your pallas kernel here")


# speedup vs baseline: 1.5489x; 1.5489x over previous
import math

import numpy as np
import jax
import jax.numpy as jnp
from jax import lax
from jax.experimental import pallas as pl
from jax.experimental.pallas import tpu as pltpu

SAMPLE_RATE = 4000
N_FFT = 64
HOP_LENGTH = N_FFT // 2
N_MELS = 16
F_MIN = 0.0
F_MAX = SAMPLE_RATE / 2.0
TOP_DB = 80.0
AMIN = 1e-10
NORM_M = -20.0
NORM_S = 20.0

N_FREQ = N_FFT // 2 + 1
N_PACK = N_FFT
LOG10_MUL = 10.0 / math.log(10.0)
INV_NORM_S = 1.0 / NORM_S


def _hann_window(n):
    k = np.arange(n, dtype=np.float64)
    return 0.5 * (1.0 - np.cos(2.0 * np.pi * k / n))


def _packed_dft():
    n = np.arange(N_FFT, dtype=np.float64)[:, None]
    k = np.arange(N_FREQ, dtype=np.float64)[None, :]
    ang = 2.0 * np.pi * n * k / N_FFT
    w = _hann_window(N_FFT)[:, None]
    cosw = w * np.cos(ang)
    sinw = -w * np.sin(ang)
    wfull = np.concatenate([cosw, sinw[:, 1:32]], axis=1).astype(np.float32)
    wsplit = np.concatenate([wfull[:HOP_LENGTH], wfull[HOP_LENGTH:]], axis=1)
    return wfull, wsplit


def _mel_filterbank():
    def hz_to_mel(f):
        return 2595.0 * np.log10(1.0 + f / 700.0)

    def mel_to_hz(m):
        return 700.0 * (10.0 ** (m / 2595.0) - 1.0)

    all_freqs = np.linspace(0.0, SAMPLE_RATE // 2, N_FREQ)
    m_pts = np.linspace(hz_to_mel(F_MIN), hz_to_mel(F_MAX), N_MELS + 2)
    f_pts = mel_to_hz(m_pts)
    f_diff = f_pts[1:] - f_pts[:-1]
    slopes = f_pts[None, :] - all_freqs[:, None]
    down = -slopes[:, :-2] / f_diff[:-1]
    up = slopes[:, 2:] / f_diff[1:]
    fb = np.maximum(0.0, np.minimum(down, up)).astype(np.float32)
    fbt = np.concatenate([fb.T, fb.T[:, 1:32]], axis=1)
    return fbt


def _make_kernel(bpb, n_rows, n_frames):
    m = bpb * n_rows

    def _body(inv_std2_ref, rows_ref, edges_ref, wsplit_ref, wfull_ref,
              fbt_ref, out_ref):
        p_all = jnp.dot(rows_ref[...], wsplit_ref[...],
                        preferred_element_type=jnp.float32)
        yb = p_all[: m - 1, :N_PACK] + p_all[1:, N_PACK:]
        ye = jnp.dot(edges_ref[...], wfull_ref[...],
                     preferred_element_type=jnp.float32)
        pb = yb * yb
        pe = ye * ye
        mel_b = jnp.einsum('mf,nf->mn', fbt_ref[...], pb,
                           preferred_element_type=jnp.float32)
        mel_e = jnp.einsum('mf,nf->mn', fbt_ref[...], pe,
                           preferred_element_type=jnp.float32)
        base = pl.program_id(0) * bpb
        for b in range(bpb):
            s2 = inv_std2_ref[base + b]
            body = mel_b[:, b * n_rows: b * n_rows + (n_frames - 2)] * s2
            edge = mel_e[:, 2 * b: 2 * b + 2] * s2
            db_body = LOG10_MUL * jnp.log(jnp.maximum(body, AMIN))
            db_edge = LOG10_MUL * jnp.log(jnp.maximum(edge, AMIN))
            lo = jnp.maximum(jnp.max(db_body), jnp.max(db_edge)) - TOP_DB
            db_body = jnp.maximum(db_body, lo)
            db_edge = jnp.maximum(db_edge, lo)
            out_ref[b, :, 1:n_frames - 1] = (db_body - NORM_M) * INV_NORM_S
            out_ref[b, :, 0:1] = (db_edge[:, 0:1] - NORM_M) * INV_NORM_S
            out_ref[b, :, n_frames - 1:n_frames] = \
                (db_edge[:, 1:2] - NORM_M) * INV_NORM_S
    return _body


def kernel(x):
    B, T = x.shape
    x = x.astype(jnp.float32)
    n_rows = T // HOP_LENGTH
    n_frames = T // HOP_LENGTH + 1

    var = jnp.var(x, axis=1, ddof=1)
    inv_std2 = (1.0 / var).astype(jnp.float32)

    rows = x.reshape(B * n_rows, HOP_LENGTH)

    pad = N_FFT // 2
    e0 = jnp.concatenate([x[:, pad:0:-1], x[:, :pad]], axis=1)
    e1 = jnp.concatenate([x[:, T - pad:], x[:, T - 2:T - pad - 2:-1]],
                         axis=1)
    edges = jnp.stack([e0, e1], axis=1).reshape(B * 2, N_FFT)

    wfull, wsplit = _packed_dft()
    fbt = _mel_filterbank()
    wfull = jnp.asarray(wfull)
    wsplit = jnp.asarray(wsplit)
    fbt = jnp.asarray(fbt)

    bpb = 8
    while B % bpb:
        bpb //= 2
    num_blocks = max(B // bpb, 1)
    m_block = bpb * n_rows

    flops = (2 * B * n_rows * HOP_LENGTH * 2 * N_PACK
             + 2 * B * n_rows * N_PACK * N_MELS
             + 6 * B * n_rows * N_PACK)
    bytes_accessed = (B * T * 4 + B * 2 * N_FFT * 4
                      + (HOP_LENGTH * 2 * N_PACK + N_FFT * N_PACK
                         + N_MELS * N_PACK) * 4
                      + B * 4 + B * N_MELS * n_frames * 4)

    out = pl.pallas_call(
        _make_kernel(bpb, n_rows, n_frames),
        out_shape=jax.ShapeDtypeStruct((B, N_MELS, n_frames), jnp.float32),
        grid_spec=pltpu.PrefetchScalarGridSpec(
            num_scalar_prefetch=1,
            grid=(num_blocks,),
            in_specs=[
                pl.BlockSpec((m_block, HOP_LENGTH), lambda i, s: (i, 0)),
                pl.BlockSpec((2 * bpb, N_FFT), lambda i, s: (i, 0)),
                pl.BlockSpec((HOP_LENGTH, 2 * N_PACK), lambda i, s: (0, 0)),
                pl.BlockSpec((N_FFT, N_PACK), lambda i, s: (0, 0)),
                pl.BlockSpec((N_MELS, N_PACK), lambda i, s: (0, 0)),
            ],
            out_specs=pl.BlockSpec((bpb, N_MELS, n_frames),
                                   lambda i, s: (i, 0, 0)),
        ),
        compiler_params=pltpu.CompilerParams(
            dimension_semantics=("parallel",)),
        cost_estimate=pl.CostEstimate(
            flops=int(flops),
            transcendentals=int(B * n_frames * N_MELS),
            bytes_accessed=int(bytes_accessed)),
    )(inv_std2, rows, edges, wsplit, wfull, fbt)

    return out
